# Initial kernel scaffold; baseline (speedup 1.0000x reference)
import jax
import jax.numpy as jnp
from jax.experimental import pallas as pl


def kernel(prediction, target, W, K_model, Kfc):
    raise NotImplementedError("write your pallas kernel here")


# speedup vs baseline: 19.8049x; 19.8049x over previous
import jax
import jax.numpy as jnp
import numpy as np
from jax.experimental import pallas as pl
from jax.experimental.pallas import tpu as pltpu

_DATA_STD = np.array([0.229, 0.224, 0.225], dtype=np.float32)
_DATA_SCALING = float(1.0 / _DATA_STD.min())


def _margin_ratio_kernel(pred_ref, w_ref, k_ref, out_ref):
    pred = pred_ref[...]
    W = w_ref[...]
    K = k_ref[0, 0]

    B, C = pred.shape

    nsq = jnp.sum(W * W, axis=1, keepdims=True)
    n = jnp.sqrt(nsq)

    m = jnp.max(pred, axis=1, keepdims=True)
    iota = jax.lax.broadcasted_iota(jnp.int32, (B, C), 1)
    eq = pred == m
    j0 = jnp.min(jnp.where(eq, iota, C), axis=1, keepdims=True)
    onehot = (iota == j0).astype(jnp.float32)

    Wj = jax.lax.dot(onehot, W, preferred_element_type=jnp.float32)
    G = jax.lax.dot_general(Wj, W, (((1,), (1,)), ((), ())),
                            preferred_element_type=jnp.float32)
    nj = jax.lax.dot(onehot, n, preferred_element_type=jnp.float32)

    S = G / (nj * n.reshape(1, C))
    dist2 = jnp.maximum(2.0 - 2.0 * S, 0.0)
    Kij = K * jnp.sqrt(dist2)

    margins = m - pred
    margins = jnp.where(eq & (iota == j0), jnp.inf, margins)
    ratios = margins / Kij
    ratio = jnp.min(ratios, axis=1)
    out_ref[0, 0] = jnp.sum(ratio) / B


def kernel(prediction, target, W, K_model, Kfc):
    del target
    K = (K_model / Kfc * _DATA_SCALING).astype(jnp.float32).reshape(1, 1)
    out = pl.pallas_call(
        _margin_ratio_kernel,
        out_shape=jax.ShapeDtypeStruct((1, 1), jnp.float32),
        in_specs=[
            pl.BlockSpec(memory_space=pltpu.VMEM),
            pl.BlockSpec(memory_space=pltpu.VMEM),
            pl.BlockSpec(memory_space=pltpu.SMEM),
        ],
        out_specs=pl.BlockSpec(memory_space=pltpu.SMEM),
    )(prediction, W, K)
    return out[0, 0]


# speedup vs baseline: 22.0132x; 1.1115x over previous
import jax
import jax.numpy as jnp
import numpy as np
from jax.experimental import pallas as pl
from jax.experimental.pallas import tpu as pltpu

_DATA_STD = np.array([0.229, 0.224, 0.225], dtype=np.float32)
_DATA_SCALING = float(1.0 / _DATA_STD.min())


def _margin_ratio_kernel(pred_ref, w_ref, k_ref, out_ref):
    pred = pred_ref[...]
    W = w_ref[...]
    K = k_ref[0, 0]

    B, C = pred.shape

    rn = jax.lax.rsqrt(jnp.sum(W * W, axis=1, keepdims=True))
    Wn = W * rn

    m = jnp.max(pred, axis=1, keepdims=True)
    iota = jax.lax.broadcasted_iota(jnp.int32, (B, C), 1)
    j0 = jnp.min(jnp.where(pred == m, iota, C), axis=1, keepdims=True)
    onehot_b = iota == j0
    onehot = onehot_b.astype(jnp.float32)

    Wjn = jax.lax.dot(onehot, Wn, preferred_element_type=jnp.float32)
    S = jax.lax.dot_general(Wjn, Wn, (((1,), (1,)), ((), ())),
                            preferred_element_type=jnp.float32)

    dist2 = jnp.maximum(2.0 - 2.0 * S, 0.0)
    margins = jnp.where(onehot_b, jnp.inf, m - pred)
    ratios = margins * jax.lax.rsqrt(dist2)
    ratio = jnp.min(ratios, axis=1)
    out_ref[0, 0] = jnp.sum(ratio) / (B * K)


def kernel(prediction, target, W, K_model, Kfc):
    del target
    K = (K_model / Kfc * _DATA_SCALING).astype(jnp.float32).reshape(1, 1)
    out = pl.pallas_call(
        _margin_ratio_kernel,
        out_shape=jax.ShapeDtypeStruct((1, 1), jnp.float32),
        in_specs=[
            pl.BlockSpec(memory_space=pltpu.VMEM),
            pl.BlockSpec(memory_space=pltpu.VMEM),
            pl.BlockSpec(memory_space=pltpu.SMEM),
        ],
        out_specs=pl.BlockSpec(memory_space=pltpu.SMEM),
    )(prediction, W, K)
    return out[0, 0]
